# Initial kernel scaffold; baseline (speedup 1.0000x reference)
import jax
import jax.numpy as jnp
from jax.experimental import pallas as pl


def kernel(m, root, edge_index):
    raise NotImplementedError("write your pallas kernel here")


# speedup vs baseline: 4.6446x; 4.6446x over previous
import functools

import jax
import jax.numpy as jnp
from jax import lax
from jax.experimental import pallas as pl
from jax.experimental.pallas import tpu as pltpu
from jax.experimental.pallas import tpu_sc as plsc

N_NODES = 10000
N_EDGES = 320000
D = 128

NC = 2
NS = 16
NW = NC * NS

E_PER_W = N_EDGES // NW
CHUNK = 80
N_CHUNKS = E_PER_W // CHUNK


def _sc_gather(m_hbm, root_hbm, idx_hbm, out_m, out_root,
               idx_v, m_rows, root_rows, sem_m, sem_r):
    wid = lax.axis_index("s") * NC + lax.axis_index("c")
    base = wid * E_PER_W
    pltpu.sync_copy(idx_hbm.at[wid], idx_v)

    def body(j, carry):
        cm = pltpu.async_copy(m_hbm.at[idx_v.at[j]], m_rows, sem_m)
        cr = pltpu.async_copy(root_hbm.at[idx_v.at[j]], root_rows, sem_r)
        cm.wait()
        pltpu.sync_copy(m_rows, out_m.at[pl.ds(base + j * CHUNK, CHUNK)])
        cr.wait()
        pltpu.sync_copy(root_rows, out_root.at[pl.ds(base + j * CHUNK, CHUNK)])
        return carry

    lax.fori_loop(0, N_CHUNKS, body, 0)


@jax.jit
def kernel(m, root, edge_index):
    src = edge_index[0].astype(jnp.int32).reshape(NW, N_CHUNKS, CHUNK)
    mesh = plsc.VectorSubcoreMesh(core_axis_name="c", subcore_axis_name="s")
    out_ty = (jax.ShapeDtypeStruct((N_EDGES, D), jnp.float32),
              jax.ShapeDtypeStruct((N_EDGES, D), jnp.float32))
    f = pl.kernel(
        _sc_gather,
        mesh=mesh,
        out_type=out_ty,
        scratch_types=[
            pltpu.VMEM((N_CHUNKS, CHUNK), jnp.int32),
            pltpu.VMEM((CHUNK, D), jnp.float32),
            pltpu.VMEM((CHUNK, D), jnp.float32),
            pltpu.SemaphoreType.DMA,
            pltpu.SemaphoreType.DMA,
        ],
    )
    return f(m, root, src)


# speedup vs baseline: 5.9728x; 1.2860x over previous
import jax
import jax.numpy as jnp
from jax import lax
from jax.experimental import pallas as pl
from jax.experimental.pallas import tpu as pltpu
from jax.experimental.pallas import tpu_sc as plsc

N_NODES = 10000
N_EDGES = 320000
D = 128

NC = 2
NS = 16
NW = NC * NS

E_PER_W = N_EDGES // NW
CHUNK = 80
N_CHUNKS = E_PER_W // CHUNK


def _sc_gather(m_hbm, root_hbm, idx_hbm, out_m, out_root,
               idx_v, m_a, r_a, m_b, r_b,
               sgm_a, sgr_a, sgm_b, sgr_b, swm_a, swr_a, swm_b, swr_b):
    wid = lax.axis_index("s") * NC + lax.axis_index("c")
    base = wid * E_PER_W
    pltpu.sync_copy(idx_hbm.at[wid], idx_v)

    def fire_gather(j, mbuf, rbuf, sm, sr):
        pltpu.make_async_copy(m_hbm.at[idx_v.at[j]], mbuf, sm).start()
        pltpu.make_async_copy(root_hbm.at[idx_v.at[j]], rbuf, sr).start()

    def wait_gather(mbuf, rbuf, sm, sr):
        pltpu.make_async_copy(m_hbm.at[idx_v.at[0]], mbuf, sm).wait()
        pltpu.make_async_copy(root_hbm.at[idx_v.at[0]], rbuf, sr).wait()

    def fire_write(j, mbuf, rbuf, sm, sr):
        dst_m = out_m.at[pl.ds(base + j * CHUNK, CHUNK)]
        dst_r = out_root.at[pl.ds(base + j * CHUNK, CHUNK)]
        pltpu.make_async_copy(mbuf, dst_m, sm).start()
        pltpu.make_async_copy(rbuf, dst_r, sr).start()

    def wait_write(mbuf, rbuf, sm, sr):
        dst_m = out_m.at[pl.ds(base, CHUNK)]
        dst_r = out_root.at[pl.ds(base, CHUNK)]
        pltpu.make_async_copy(mbuf, dst_m, sm).wait()
        pltpu.make_async_copy(rbuf, dst_r, sr).wait()

    fire_gather(0, m_a, r_a, sgm_a, sgr_a)
    fire_gather(1, m_b, r_b, sgm_b, sgr_b)
    wait_gather(m_a, r_a, sgm_a, sgr_a)
    fire_write(0, m_a, r_a, swm_a, swr_a)
    wait_write(m_a, r_a, swm_a, swr_a)
    fire_gather(2, m_a, r_a, sgm_a, sgr_a)
    wait_gather(m_b, r_b, sgm_b, sgr_b)
    fire_write(1, m_b, r_b, swm_b, swr_b)

    def body(kk, carry):
        wait_write(m_b, r_b, swm_b, swr_b)
        fire_gather(2 * kk + 1, m_b, r_b, sgm_b, sgr_b)
        wait_gather(m_a, r_a, sgm_a, sgr_a)
        fire_write(2 * kk, m_a, r_a, swm_a, swr_a)
        wait_write(m_a, r_a, swm_a, swr_a)
        fire_gather(2 * kk + 2, m_a, r_a, sgm_a, sgr_a)
        wait_gather(m_b, r_b, sgm_b, sgr_b)
        fire_write(2 * kk + 1, m_b, r_b, swm_b, swr_b)
        return carry

    lax.fori_loop(1, (N_CHUNKS - 1) // 2, body, 0)

    wait_write(m_b, r_b, swm_b, swr_b)
    wait_gather(m_a, r_a, sgm_a, sgr_a)
    fire_write(N_CHUNKS - 1, m_a, r_a, swm_a, swr_a)
    wait_write(m_a, r_a, swm_a, swr_a)


@jax.jit
def kernel(m, root, edge_index):
    src = edge_index[0].astype(jnp.int32).reshape(NW, N_CHUNKS, CHUNK)
    mesh = plsc.VectorSubcoreMesh(core_axis_name="c", subcore_axis_name="s")
    out_ty = (jax.ShapeDtypeStruct((N_EDGES, D), jnp.float32),
              jax.ShapeDtypeStruct((N_EDGES, D), jnp.float32))
    f = pl.kernel(
        _sc_gather,
        mesh=mesh,
        out_type=out_ty,
        scratch_types=[
            pltpu.VMEM((N_CHUNKS, CHUNK), jnp.int32),
            pltpu.VMEM((CHUNK, D), jnp.float32),
            pltpu.VMEM((CHUNK, D), jnp.float32),
            pltpu.VMEM((CHUNK, D), jnp.float32),
            pltpu.VMEM((CHUNK, D), jnp.float32),
        ] + [pltpu.SemaphoreType.DMA] * 8,
    )
    return f(m, root, src)


# speedup vs baseline: 6.0093x; 1.0061x over previous
import jax
import jax.numpy as jnp
from jax import lax
from jax.experimental import pallas as pl
from jax.experimental.pallas import tpu as pltpu
from jax.experimental.pallas import tpu_sc as plsc

N_NODES = 10000
N_EDGES = 320000
D = 128

NC = 2
NS = 16
NW = NC * NS

E_PER_W = N_EDGES // NW
CHUNK = 80
N_CHUNKS = E_PER_W // CHUNK
S = 5
G = 3
N_GROUPS = N_CHUNKS // S


def _sc_gather(m_hbm, root_hbm, idx_hbm, out_m, out_root, idx_v, *rest):
    bm = rest[0:S]
    br = rest[S:2 * S]
    sgm = rest[2 * S:3 * S]
    sgr = rest[3 * S:4 * S]
    swm = rest[4 * S:5 * S]
    swr = rest[5 * S:6 * S]

    wid = lax.axis_index("s") * NC + lax.axis_index("c")
    base = wid * E_PER_W
    pltpu.sync_copy(idx_hbm.at[wid], idx_v)

    def fire_gather(j, s):
        pltpu.make_async_copy(m_hbm.at[idx_v.at[j]], bm[s], sgm[s]).start()
        pltpu.make_async_copy(root_hbm.at[idx_v.at[j]], br[s], sgr[s]).start()

    def wait_gather(s):
        pltpu.make_async_copy(m_hbm.at[idx_v.at[0]], bm[s], sgm[s]).wait()
        pltpu.make_async_copy(root_hbm.at[idx_v.at[0]], br[s], sgr[s]).wait()

    def fire_write(j, s):
        dst_m = out_m.at[pl.ds(base + j * CHUNK, CHUNK)]
        dst_r = out_root.at[pl.ds(base + j * CHUNK, CHUNK)]
        pltpu.make_async_copy(bm[s], dst_m, swm[s]).start()
        pltpu.make_async_copy(br[s], dst_r, swr[s]).start()

    def wait_write(s):
        dst_m = out_m.at[pl.ds(base, CHUNK)]
        dst_r = out_root.at[pl.ds(base, CHUNK)]
        pltpu.make_async_copy(bm[s], dst_m, swm[s]).wait()
        pltpu.make_async_copy(br[s], dst_r, swr[s]).wait()

    for j in range(G):
        fire_gather(j, j)
    for k in range(S):
        wait_gather(k)
        fire_write(k, k)
        s3 = (k + G) % S
        if k + G >= S:
            wait_write(s3)
        fire_gather(k + G, s3)

    def body(g, carry):
        for i in range(S):
            k = g * S + i
            wait_gather(i)
            fire_write(k, i)
            s3 = (i + G) % S
            wait_write(s3)
            fire_gather(k + G, s3)
        return carry

    lax.fori_loop(1, N_GROUPS - 1, body, 0)

    for i in range(S):
        k = (N_GROUPS - 1) * S + i
        wait_gather(i)
        fire_write(k, i)
        if k + G < N_CHUNKS:
            s3 = (i + G) % S
            wait_write(s3)
            fire_gather(k + G, s3)
    for i in range(S):
        wait_write(i)


@jax.jit
def kernel(m, root, edge_index):
    src = edge_index[0].astype(jnp.int32).reshape(NW, N_CHUNKS, CHUNK)
    mesh = plsc.VectorSubcoreMesh(core_axis_name="c", subcore_axis_name="s")
    out_ty = (jax.ShapeDtypeStruct((N_EDGES, D), jnp.float32),
              jax.ShapeDtypeStruct((N_EDGES, D), jnp.float32))
    f = pl.kernel(
        _sc_gather,
        mesh=mesh,
        out_type=out_ty,
        scratch_types=[
            pltpu.VMEM((N_CHUNKS, CHUNK), jnp.int32),
        ] + [pltpu.VMEM((CHUNK, D), jnp.float32)] * (2 * S)
          + [pltpu.SemaphoreType.DMA] * (4 * S),
    )
    return f(m, root, src)
